# baseline (speedup 1.0000x reference)
import functools

import jax
import jax.numpy as jnp
from jax import lax
from jax.experimental import pallas as pl
from jax.experimental.pallas import tpu as pltpu
from jax.experimental.pallas import tpu_sc as plsc

N = 10000
NP = 10112
F = 64
E = 320000
NC = 2
NS = 16
NW = NC * NS
CHUNK = 128
NCH = 81
NBUF = 3
EPT = NCH * CHUNK
EPAD = NW * EPT
RPT = NP // NS

_mesh = plsc.VectorSubcoreMesh(core_axis_name="c", subcore_axis_name="s")
_sc_params = pltpu.CompilerParams(use_tc_tiling_on_sc=False)


@functools.partial(
    pl.kernel,
    out_type=jax.ShapeDtypeStruct((NC, NP, F), jnp.float32),
    mesh=_mesh,
    scratch_types=[
        pltpu.VMEM((NCH, CHUNK), jnp.int32),
        pltpu.VMEM((NCH, CHUNK), jnp.int32),
        pltpu.VMEM((NBUF, CHUNK, F), jnp.float32),
        pltpu.VMEM_SHARED((NP, F), jnp.float32),
        pltpu.VMEM_SHARED((NP, F), jnp.float32),
        pltpu.SemaphoreType.DMA,
        pltpu.SemaphoreType.DMA,
        pltpu.SemaphoreType.DMA,
        pltpu.SemaphoreType.DMA,
    ],
    compiler_params=_sc_params,
)
def _hop(g_hbm, src_hbm, dst_hbm, zeros_hbm, out_hbm,
         src_v, dst_v, rows_v, g_sh, agg_sh, sem0, sem1, sem2, sem3):
    sems = [sem0, sem1, sem2, sem3]
    c = lax.axis_index("c")
    s = lax.axis_index("s")
    wid = s * NC + c
    pltpu.sync_copy(zeros_hbm.at[pl.ds(s * RPT, RPT)],
                    agg_sh.at[pl.ds(s * RPT, RPT)])
    pltpu.sync_copy(g_hbm.at[pl.ds(s * RPT, RPT)],
                    g_sh.at[pl.ds(s * RPT, RPT)])
    pltpu.sync_copy(src_hbm.at[wid], src_v)
    pltpu.sync_copy(dst_hbm.at[wid], dst_v)
    plsc.subcore_barrier()

    def gather(j, b):
        pltpu.async_copy(g_sh.at[src_v.at[j]], rows_v.at[b], sems[b])

    def gather_wait(b):
        pltpu.make_async_copy(g_sh.at[src_v.at[0]], rows_v.at[b],
                              sems[b]).wait()

    for b in range(NBUF):
        gather(b, b)

    def body(j4, carry):
        for b in range(NBUF):
            j = j4 * NBUF + b
            gather_wait(b)
            pltpu.sync_copy(rows_v.at[b], agg_sh.at[dst_v.at[j]], add=True)

            @pl.when(j + NBUF < NCH)
            def _():
                gather(j + NBUF, b)
        return carry

    lax.fori_loop(0, NCH // NBUF, body, 0)
    plsc.subcore_barrier()
    pltpu.sync_copy(agg_sh.at[pl.ds(s * RPT, RPT)],
                    out_hbm.at[c, pl.ds(s * RPT, RPT)])


@functools.partial(
    pl.kernel,
    out_type=jax.ShapeDtypeStruct((NC, 3, NP, 16), jnp.float32),
    mesh=_mesh,
    scratch_types=[
        pltpu.VMEM((NCH, CHUNK), jnp.int32),
        pltpu.VMEM((CHUNK, 16), jnp.float32),
        pltpu.VMEM_SHARED((NP, 16), jnp.float32),
        pltpu.SemaphoreType.DMA,
    ],
    compiler_params=_sc_params,
)
def _deg(dst3_hbm, ones_hbm, zeros_hbm, out_hbm, dst_v, ones_v, dega_sh, sem):
    c = lax.axis_index("c")
    s = lax.axis_index("s")
    wid = s * NC + c
    pltpu.sync_copy(ones_hbm, ones_v)
    for r in range(3):
        pltpu.sync_copy(zeros_hbm.at[pl.ds(s * RPT, RPT)],
                        dega_sh.at[pl.ds(s * RPT, RPT)])
        pltpu.sync_copy(dst3_hbm.at[r, wid], dst_v)
        plsc.subcore_barrier()

        def body(j, carry):
            pltpu.sync_copy(ones_v, dega_sh.at[dst_v.at[j]], add=True)
            return carry

        lax.fori_loop(0, NCH, body, 0)
        plsc.subcore_barrier()
        pltpu.sync_copy(dega_sh.at[pl.ds(s * RPT, RPT)],
                        out_hbm.at[c, r, pl.ds(s * RPT, RPT)])


def _act(x):
    return jnp.where(x > 0, x, 0.01 * x)


def _mlp_body(x_ref, w1_ref, b1_ref, w2_ref, b2_ref, o_ref):
    h = jnp.dot(x_ref[...], w1_ref[...], preferred_element_type=jnp.float32,
                 precision=lax.Precision.HIGHEST)
    h = _act(h + b1_ref[...])
    h = jnp.dot(h, w2_ref[...], preferred_element_type=jnp.float32,
                 precision=lax.Precision.HIGHEST)
    o_ref[...] = _act(h + b2_ref[...])


def _ug_body(degp_ref, h_ref, u3_ref, g_ref):
    us = []
    for r in range(3):
        deg = degp_ref[0, r, :, 0:1] + degp_ref[1, r, :, 0:1]
        us.append(jnp.power(jnp.maximum(deg, 1.0), -0.5))
    u3_ref[...] = jnp.concatenate(us, axis=1)
    g_ref[...] = h_ref[...] * us[0]


def _comb_body(r, p_ref, f_ref, u3_ref, f1_ref, g1_ref):
    u = u3_ref[:, r:r + 1]
    f1 = f_ref[...] - (p_ref[0] + p_ref[1]) * u
    f1_ref[...] = f1
    g1_ref[...] = f1 * u


def _poly(p_ref, f0_ref, f1_ref, u3_ref, w3_ref, b3_ref, hs_ref, r):
    f1 = f1_ref[...]
    f2 = f1 - (p_ref[0] + p_ref[1]) * u3_ref[:, r:r + 1]
    w3 = w3_ref[...]
    wa, wb, wc = w3[0:64], w3[64:128], w3[128:192]
    a0 = 3.0 * wa
    a1 = 3.0 * wb - 3.0 * wa
    a2 = 0.75 * wa - 1.5 * wb + 0.75 * wc
    h = jnp.dot(f0_ref[...], a0, preferred_element_type=jnp.float32,
                 precision=lax.Precision.HIGHEST)
    h += jnp.dot(f1, a1, preferred_element_type=jnp.float32,
                 precision=lax.Precision.HIGHEST)
    h += jnp.dot(f2, a2, preferred_element_type=jnp.float32,
                 precision=lax.Precision.HIGHEST)
    h += b3_ref[...]
    return h, hs_ref[...] + h


def _fin_mid_body(r, p_ref, f0_ref, f1_ref, u3_ref, w3_ref, b3_ref, hs_ref,
                  h_ref, hso_ref, g_ref):
    h, hs = _poly(p_ref, f0_ref, f1_ref, u3_ref, w3_ref, b3_ref, hs_ref, r)
    h_ref[...] = h
    hso_ref[...] = hs
    g_ref[...] = h * u3_ref[:, r + 1:r + 2]


def _fin_last_body(r, p_ref, f0_ref, f1_ref, u3_ref, w3_ref, b3_ref, hs_ref,
                   w4_ref, b4_ref, o_ref):
    _, hs = _poly(p_ref, f0_ref, f1_ref, u3_ref, w3_ref, b3_ref, hs_ref, r)
    o_ref[...] = jnp.dot(_act(hs), w4_ref[...],
                         preferred_element_type=jnp.float32,
                 precision=lax.Precision.HIGHEST) + b4_ref[...]


BLK = 1264
_TGRID = (NP // BLK,)


def _rows(w):
    return pl.BlockSpec((BLK, w), lambda i: (i, 0))


def _bcast(*shape):
    return pl.BlockSpec(shape, lambda i: tuple(0 for _ in shape))


_F_OUT = jax.ShapeDtypeStruct((NP, F), jnp.float32)

_mlp = pl.pallas_call(
    _mlp_body, grid=_TGRID,
    in_specs=[_rows(128), _bcast(128, F), _bcast(1, F), _bcast(F, F),
              _bcast(1, F)],
    out_specs=_rows(F), out_shape=_F_OUT)
_ug = pl.pallas_call(
    _ug_body, grid=_TGRID,
    in_specs=[pl.BlockSpec((NC, 3, BLK, 16), lambda i: (0, 0, i, 0)),
              _rows(F)],
    out_specs=(_rows(3), _rows(F)),
    out_shape=(jax.ShapeDtypeStruct((NP, 3), jnp.float32), _F_OUT))

_P_SPEC = pl.BlockSpec((NC, BLK, F), lambda i: (0, i, 0))


def _comb(r):
    return pl.pallas_call(
        functools.partial(_comb_body, r), grid=_TGRID,
        in_specs=[_P_SPEC, _rows(F), _rows(3)],
        out_specs=(_rows(F), _rows(F)),
        out_shape=(_F_OUT, _F_OUT))


def _fin_mid(r):
    return pl.pallas_call(
        functools.partial(_fin_mid_body, r), grid=_TGRID,
        in_specs=[_P_SPEC, _rows(F), _rows(F), _rows(3),
                  _bcast(3 * F, F), _bcast(1, F), _rows(F)],
        out_specs=(_rows(F), _rows(F), _rows(F)),
        out_shape=(_F_OUT, _F_OUT, _F_OUT))


_fin_last = pl.pallas_call(
    functools.partial(_fin_last_body, 2), grid=_TGRID,
    in_specs=[_P_SPEC, _rows(F), _rows(F), _rows(3),
              _bcast(3 * F, F), _bcast(1, F), _rows(F),
              _bcast(F, 128), _bcast(1, 128)],
    out_specs=_rows(128),
    out_shape=jax.ShapeDtypeStruct((NP, 128), jnp.float32))


def _prep_edges(ei):
    pad = N + jnp.arange(EPAD - E, dtype=jnp.int32) % (NP - N)
    srcp = jnp.concatenate([ei[0].astype(jnp.int32), pad]).reshape(
        NW, NCH, CHUNK)
    dstp = jnp.concatenate([ei[1].astype(jnp.int32), pad]).reshape(
        NW, NCH, CHUNK)
    return srcp, dstp


@jax.jit
def kernel(in_feat, edge_index_r0, edge_index_r1, edge_index_r2,
           W1, b1, W2, b2, W3, b3, W4, b4):
    xp = jnp.pad(in_feat, ((0, NP - N), (0, 0)))
    edges = [_prep_edges(e) for e in (edge_index_r0, edge_index_r1,
                                      edge_index_r2)]
    dst3 = jnp.stack([d for (_, d) in edges])
    z64 = jnp.zeros((NP, F), jnp.float32)
    z16 = jnp.zeros((NP, 16), jnp.float32)
    ones16 = jnp.ones((CHUNK, 16), jnp.float32)

    degp = _deg(dst3, ones16, z16)
    h = _mlp(xp, W1, b1.reshape(1, F), W2, b2.reshape(1, F))
    u3, g = _ug(degp, h)

    b3r = b3.reshape(1, F)
    w4p = jnp.pad(W4, ((0, 0), (0, 126)))
    b4p = jnp.pad(b4, (0, 126)).reshape(1, 128)

    hsum = z64
    for r, (srcp, dstp) in enumerate(edges):
        p1 = _hop(g, srcp, dstp, z64)
        f1, g1 = _comb(r)(p1, h, u3)
        p2 = _hop(g1, srcp, dstp, z64)
        if r < 2:
            h, hsum, g = _fin_mid(r)(p2, h, f1, u3, W3, b3r, hsum)
        else:
            logits = _fin_last(p2, h, f1, u3, W3, b3r, hsum, w4p, b4p)
    return logits[:N, :2]


# speedup vs baseline: 1.1547x; 1.1547x over previous
import functools

import jax
import jax.numpy as jnp
from jax import lax
from jax.experimental import pallas as pl
from jax.experimental.pallas import tpu as pltpu
from jax.experimental.pallas import tpu_sc as plsc

N = 10000
NP = 10112
F = 64
E = 320000
NC = 2
NS = 16
NW = NC * NS
CHUNK = 128
NCH = 81
NBUF = 3
EPT = NCH * CHUNK
EPAD = NW * EPT
RPT = NP // NS

_mesh = plsc.VectorSubcoreMesh(core_axis_name="c", subcore_axis_name="s")
_sc_params = pltpu.CompilerParams(use_tc_tiling_on_sc=False)


@functools.partial(
    pl.kernel,
    out_type=jax.ShapeDtypeStruct((NC, NP, F), jnp.float32),
    mesh=_mesh,
    scratch_types=[
        pltpu.VMEM((NCH, CHUNK), jnp.int32),
        pltpu.VMEM((NCH, CHUNK), jnp.int32),
        pltpu.VMEM((NBUF, CHUNK, F), jnp.float32),
        pltpu.VMEM_SHARED((NP, F), jnp.float32),
        pltpu.VMEM_SHARED((NP, F), jnp.float32),
        pltpu.SemaphoreType.DMA,
        pltpu.SemaphoreType.DMA,
        pltpu.SemaphoreType.DMA,
        pltpu.SemaphoreType.DMA,
        pltpu.SemaphoreType.DMA,
        pltpu.SemaphoreType.DMA,
    ],
    compiler_params=_sc_params,
)
def _hop(g_hbm, src_hbm, dst_hbm, zeros_hbm, out_hbm,
         src_v, dst_v, rows_v, g_sh, agg_sh,
         gsem0, gsem1, gsem2, ssem0, ssem1, ssem2):
    gsems = [gsem0, gsem1, gsem2]
    ssems = [ssem0, ssem1, ssem2]
    c = lax.axis_index("c")
    s = lax.axis_index("s")
    wid = s * NC + c
    pltpu.sync_copy(zeros_hbm.at[pl.ds(s * RPT, RPT)],
                    agg_sh.at[pl.ds(s * RPT, RPT)])
    pltpu.sync_copy(g_hbm.at[pl.ds(s * RPT, RPT)],
                    g_sh.at[pl.ds(s * RPT, RPT)])
    pltpu.sync_copy(src_hbm.at[wid], src_v)
    pltpu.sync_copy(dst_hbm.at[wid], dst_v)
    plsc.subcore_barrier()

    def gather(j, b):
        pltpu.async_copy(g_sh.at[src_v.at[j]], rows_v.at[b], gsems[b])

    def gather_wait(b):
        pltpu.make_async_copy(g_sh.at[src_v.at[0]], rows_v.at[b],
                              gsems[b]).wait()

    def scatter(j, b):
        pltpu.async_copy(rows_v.at[b], agg_sh.at[dst_v.at[j]], ssems[b],
                         add=True)

    def scatter_wait(b):
        pltpu.make_async_copy(rows_v.at[b], agg_sh.at[dst_v.at[0]],
                              ssems[b]).wait()

    for b in range(NBUF):
        gather(b, b)

    def body(j4, carry):
        for b in range(NBUF):
            j = j4 * NBUF + b
            gather_wait(b)
            scatter(j, b)
            bp = (b - 1) % NBUF
            if b == 0:
                @pl.when(j4 > 0)
                def _():
                    scatter_wait(bp)

                @pl.when((j4 > 0) & (j + NBUF - 1 < NCH))
                def _():
                    gather(j + NBUF - 1, bp)
            else:
                scatter_wait(bp)

                @pl.when(j + NBUF - 1 < NCH)
                def _():
                    gather(j + NBUF - 1, bp)
        return carry

    lax.fori_loop(0, NCH // NBUF, body, 0)
    scatter_wait((NCH - 1) % NBUF)
    plsc.subcore_barrier()
    pltpu.sync_copy(agg_sh.at[pl.ds(s * RPT, RPT)],
                    out_hbm.at[c, pl.ds(s * RPT, RPT)])


@functools.partial(
    pl.kernel,
    out_type=jax.ShapeDtypeStruct((NC, 3, NP, 16), jnp.float32),
    mesh=_mesh,
    scratch_types=[
        pltpu.VMEM((NCH, CHUNK), jnp.int32),
        pltpu.VMEM((CHUNK, 16), jnp.float32),
        pltpu.VMEM_SHARED((NP, 16), jnp.float32),
        pltpu.SemaphoreType.DMA,
    ],
    compiler_params=_sc_params,
)
def _deg(dst3_hbm, ones_hbm, zeros_hbm, out_hbm, dst_v, ones_v, dega_sh, sem):
    c = lax.axis_index("c")
    s = lax.axis_index("s")
    wid = s * NC + c
    pltpu.sync_copy(ones_hbm, ones_v)
    for r in range(3):
        pltpu.sync_copy(zeros_hbm.at[pl.ds(s * RPT, RPT)],
                        dega_sh.at[pl.ds(s * RPT, RPT)])
        pltpu.sync_copy(dst3_hbm.at[r, wid], dst_v)
        plsc.subcore_barrier()

        def body(j, carry):
            pltpu.sync_copy(ones_v, dega_sh.at[dst_v.at[j]], add=True)
            return carry

        lax.fori_loop(0, NCH, body, 0)
        plsc.subcore_barrier()
        pltpu.sync_copy(dega_sh.at[pl.ds(s * RPT, RPT)],
                        out_hbm.at[c, r, pl.ds(s * RPT, RPT)])


def _act(x):
    return jnp.where(x > 0, x, 0.01 * x)


def _dot(a, b):
    return jnp.dot(a.astype(jnp.bfloat16), b.astype(jnp.bfloat16),
                   preferred_element_type=jnp.float32)


def _mlp_body(x_ref, w1_ref, b1_ref, w2_ref, b2_ref, o_ref):
    h = _act(_dot(x_ref[...], w1_ref[...]) + b1_ref[...])
    o_ref[...] = _act(_dot(h, w2_ref[...]) + b2_ref[...])


def _ug_body(degp_ref, h_ref, u3_ref, g_ref):
    us = []
    for r in range(3):
        deg = degp_ref[0, r, :, 0:1] + degp_ref[1, r, :, 0:1]
        us.append(jnp.power(jnp.maximum(deg, 1.0), -0.5))
    u3_ref[...] = jnp.concatenate(us, axis=1)
    g_ref[...] = h_ref[...] * us[0]


def _comb_body(r, p_ref, f_ref, u3_ref, f1_ref, g1_ref):
    u = u3_ref[:, r:r + 1]
    f1 = f_ref[...] - (p_ref[0] + p_ref[1]) * u
    f1_ref[...] = f1
    g1_ref[...] = f1 * u


def _poly(p_ref, f0_ref, f1_ref, u3_ref, w3_ref, b3_ref, hs_ref, r):
    f0 = f0_ref[...]
    f1 = f1_ref[...]
    f2 = f1 - (p_ref[0] + p_ref[1]) * u3_ref[:, r:r + 1]
    w3 = w3_ref[...]
    h0 = 3.0 * f0 - 3.0 * f1 + 0.75 * f2
    h1 = 3.0 * f1 - 1.5 * f2
    h2 = 0.75 * f2
    h = _dot(h0, w3[0:64]) + _dot(h1, w3[64:128]) + _dot(h2, w3[128:192])
    h += b3_ref[...]
    return h, hs_ref[...] + h


def _fin_mid_body(r, p_ref, f0_ref, f1_ref, u3_ref, w3_ref, b3_ref, hs_ref,
                  h_ref, hso_ref, g_ref):
    h, hs = _poly(p_ref, f0_ref, f1_ref, u3_ref, w3_ref, b3_ref, hs_ref, r)
    h_ref[...] = h
    hso_ref[...] = hs
    g_ref[...] = h * u3_ref[:, r + 1:r + 2]


def _fin_last_body(r, p_ref, f0_ref, f1_ref, u3_ref, w3_ref, b3_ref, hs_ref,
                   w4_ref, b4_ref, o_ref):
    _, hs = _poly(p_ref, f0_ref, f1_ref, u3_ref, w3_ref, b3_ref, hs_ref, r)
    o_ref[...] = _dot(_act(hs), w4_ref[...]) + b4_ref[...]


BLK = 1264
_TGRID = (NP // BLK,)


def _rows(w):
    return pl.BlockSpec((BLK, w), lambda i: (i, 0))


def _bcast(*shape):
    return pl.BlockSpec(shape, lambda i: tuple(0 for _ in shape))


_F_OUT = jax.ShapeDtypeStruct((NP, F), jnp.float32)

_mlp = pl.pallas_call(
    _mlp_body, grid=_TGRID,
    in_specs=[_rows(128), _bcast(128, F), _bcast(1, F), _bcast(F, F),
              _bcast(1, F)],
    out_specs=_rows(F), out_shape=_F_OUT)
_ug = pl.pallas_call(
    _ug_body, grid=_TGRID,
    in_specs=[pl.BlockSpec((NC, 3, BLK, 16), lambda i: (0, 0, i, 0)),
              _rows(F)],
    out_specs=(_rows(3), _rows(F)),
    out_shape=(jax.ShapeDtypeStruct((NP, 3), jnp.float32), _F_OUT))

_P_SPEC = pl.BlockSpec((NC, BLK, F), lambda i: (0, i, 0))


def _comb(r):
    return pl.pallas_call(
        functools.partial(_comb_body, r), grid=_TGRID,
        in_specs=[_P_SPEC, _rows(F), _rows(3)],
        out_specs=(_rows(F), _rows(F)),
        out_shape=(_F_OUT, _F_OUT))


def _fin_mid(r):
    return pl.pallas_call(
        functools.partial(_fin_mid_body, r), grid=_TGRID,
        in_specs=[_P_SPEC, _rows(F), _rows(F), _rows(3),
                  _bcast(3 * F, F), _bcast(1, F), _rows(F)],
        out_specs=(_rows(F), _rows(F), _rows(F)),
        out_shape=(_F_OUT, _F_OUT, _F_OUT))


_fin_last = pl.pallas_call(
    functools.partial(_fin_last_body, 2), grid=_TGRID,
    in_specs=[_P_SPEC, _rows(F), _rows(F), _rows(3),
              _bcast(3 * F, F), _bcast(1, F), _rows(F),
              _bcast(F, 128), _bcast(1, 128)],
    out_specs=_rows(128),
    out_shape=jax.ShapeDtypeStruct((NP, 128), jnp.float32))


def _prep_edges(ei):
    pad = N + jnp.arange(EPAD - E, dtype=jnp.int32) % (NP - N)
    srcp = jnp.concatenate([ei[0].astype(jnp.int32), pad]).reshape(
        NW, NCH, CHUNK)
    dstp = jnp.concatenate([ei[1].astype(jnp.int32), pad]).reshape(
        NW, NCH, CHUNK)
    return srcp, dstp


@jax.jit
def kernel(in_feat, edge_index_r0, edge_index_r1, edge_index_r2,
           W1, b1, W2, b2, W3, b3, W4, b4):
    xp = jnp.pad(in_feat, ((0, NP - N), (0, 0)))
    edges = [_prep_edges(e) for e in (edge_index_r0, edge_index_r1,
                                      edge_index_r2)]
    dst3 = jnp.stack([d for (_, d) in edges])
    z64 = jnp.zeros((NP, F), jnp.float32)
    z16 = jnp.zeros((NP, 16), jnp.float32)
    ones16 = jnp.ones((CHUNK, 16), jnp.float32)

    degp = _deg(dst3, ones16, z16)
    h = _mlp(xp, W1, b1.reshape(1, F), W2, b2.reshape(1, F))
    u3, g = _ug(degp, h)

    b3r = b3.reshape(1, F)
    w4p = jnp.pad(W4, ((0, 0), (0, 126)))
    b4p = jnp.pad(b4, (0, 126)).reshape(1, 128)

    hsum = z64
    for r, (srcp, dstp) in enumerate(edges):
        p1 = _hop(g, srcp, dstp, z64)
        f1, g1 = _comb(r)(p1, h, u3)
        p2 = _hop(g1, srcp, dstp, z64)
        if r < 2:
            h, hsum, g = _fin_mid(r)(p2, h, f1, u3, W3, b3r, hsum)
        else:
            logits = _fin_last(p2, h, f1, u3, W3, b3r, hsum, w4p, b4p)
    return logits[:N, :2]
